# baseline (speedup 1.0000x reference)
import functools

import jax
import jax.numpy as jnp
from jax import lax
from jax.experimental import pallas as pl
from jax.experimental.pallas import tpu as pltpu
from jax.experimental.pallas import tpu_sc as plsc

N = 10000
E = 320000
D = 128
NCLS = 16

_NUM_CORES = 2
_NUM_SUBCORES = 16
_NW = _NUM_CORES * _NUM_SUBCORES
_EPT = E // _NW
_CHUNK = 80
_NBUF = 3
_GA = 2
_LAG = _NBUF - _GA
_SLABCH = 25
_SLABPAD = 32
_NSG = _EPT // (_SLABCH * _CHUNK)
_ZSLAB = 200
_NSLAB = N // _ZSLAB


@functools.cache
def _sc_gather_scatter_kernel():
    mesh = plsc.VectorSubcoreMesh(core_axis_name="c", subcore_axis_name="s")

    @functools.partial(
        pl.kernel,
        mesh=mesh,
        out_type=jax.ShapeDtypeStruct((_NUM_CORES, N, D), jnp.float32),
        scratch_types=[
            pltpu.VMEM((2 * _SLABPAD, _CHUNK), jnp.int32),
            pltpu.VMEM((2 * _SLABPAD, _CHUNK), jnp.int32),
            pltpu.VMEM((_NBUF * _CHUNK, D), jnp.float32),
            pltpu.VMEM_SHARED((N, D), jnp.float32),
            pltpu.SemaphoreType.DMA,
            pltpu.SemaphoreType.DMA,
            pltpu.SemaphoreType.DMA,
        ],
    )
    def body(hw_hbm, src_hbm, dst_hbm, out_hbm, src_v, dst_v, rows_v, acc,
             gsem, ssem, isem):
        c = lax.axis_index("c")
        s = lax.axis_index("s")
        wid = c * _NUM_SUBCORES + s

        def zero_rows(t, carry):
            rows_v[t // (D // 16), pl.ds((t % (D // 16)) * 16, 16)] = (
                jnp.zeros((16,), jnp.float32))
            return carry

        lax.fori_loop(0, _ZSLAB * (D // 16), zero_rows, 0)

        nz = (_NSLAB + _NUM_SUBCORES - 1) // _NUM_SUBCORES
        for k in range(nz):
            j = s + k * _NUM_SUBCORES

            @pl.when(j < _NSLAB)
            def _():
                pltpu.async_copy(rows_v.at[pl.ds(0, _ZSLAB)],
                                 acc.at[pl.ds(j * _ZSLAB, _ZSLAB)], isem)

        for k in range(nz):
            j = s + k * _NUM_SUBCORES

            @pl.when(j < _NSLAB)
            def _():
                pltpu.make_async_copy(rows_v.at[pl.ds(0, _ZSLAB)],
                                      acc.at[pl.ds(0, _ZSLAB)], isem).wait()

        plsc.subcore_barrier()

        ebase = wid * _NSG
        nch = _SLABCH * _NSG

        def _irow(j):
            sg = j // _SLABCH
            return lax.rem(sg, 2) * _SLABPAD + (j - sg * _SLABCH)

        def _fire_gather(j):
            pltpu.async_copy(
                hw_hbm.at[src_v.at[_irow(j)]],
                rows_v.at[pl.ds(lax.rem(j, _NBUF) * _CHUNK, _CHUNK)], gsem)

        def _fire_scatter(j):
            pltpu.async_copy(
                rows_v.at[pl.ds(lax.rem(j, _NBUF) * _CHUNK, _CHUNK)],
                acc.at[dst_v.at[_irow(j)]], ssem, add=True)

        def _drain(sem, n):
            for _ in range(n):
                pltpu.make_async_copy(
                    hw_hbm.at[src_v.at[0]], rows_v.at[pl.ds(0, _CHUNK)],
                    sem).wait()

        pltpu.async_copy(src_hbm.at[ebase], src_v.at[pl.ds(0, _SLABPAD)],
                         isem).wait()
        pltpu.async_copy(dst_hbm.at[ebase], dst_v.at[pl.ds(0, _SLABPAD)],
                         isem).wait()
        pltpu.async_copy(src_hbm.at[ebase + 1],
                         src_v.at[pl.ds(_SLABPAD, _SLABPAD)], isem)
        pltpu.async_copy(dst_hbm.at[ebase + 1],
                         dst_v.at[pl.ds(_SLABPAD, _SLABPAD)], isem)
        for j0 in range(_GA):
            _fire_gather(j0)

        def step(j, carry):
            @pl.when((lax.rem(j + _GA, _SLABCH) == 0)
                     & (j + _GA > 0) & (j + _GA < nch))
            def _():
                sgn = (j + _GA) // _SLABCH
                base = lax.rem(sgn, 2) * _SLABPAD
                pltpu.make_async_copy(
                    src_hbm.at[ebase], src_v.at[pl.ds(base, _SLABPAD)],
                    isem).wait()
                pltpu.make_async_copy(
                    dst_hbm.at[ebase], dst_v.at[pl.ds(base, _SLABPAD)],
                    isem).wait()

            @pl.when((lax.rem(j, _SLABCH) == 2) & (j > _SLABCH)
                     & (j < (_NSG - 1) * _SLABCH))
            def _():
                sgp = j // _SLABCH + 1
                base = lax.rem(sgp, 2) * _SLABPAD
                pltpu.async_copy(src_hbm.at[ebase + sgp],
                                 src_v.at[pl.ds(base, _SLABPAD)], isem)
                pltpu.async_copy(dst_hbm.at[ebase + sgp],
                                 dst_v.at[pl.ds(base, _SLABPAD)], isem)

            @pl.when(j < nch)
            def _():
                _drain(gsem, 1)
                _fire_scatter(j)

            @pl.when((j >= _LAG) & (j < nch + _LAG))
            def _():
                _drain(ssem, 1)

            @pl.when(j + _GA < nch)
            def _():
                _fire_gather(j + _GA)

            return carry

        lax.fori_loop(0, nch + 2, step, 0)
        plsc.subcore_barrier()

        ndump = 0
        for k in range((_NSLAB + _NUM_SUBCORES - 1) // _NUM_SUBCORES):
            j = s + k * _NUM_SUBCORES

            @pl.when(j < _NSLAB)
            def _():
                pltpu.async_copy(acc.at[pl.ds(j * _ZSLAB, _ZSLAB)],
                                 out_hbm.at[c, pl.ds(j * _ZSLAB, _ZSLAB)],
                                 isem)

            ndump += 1

        for k in range(ndump):
            j = s + k * _NUM_SUBCORES

            @pl.when(j < _NSLAB)
            def _():
                pltpu.make_async_copy(
                    acc.at[pl.ds(0, _ZSLAB)],
                    out_hbm.at[c, pl.ds(0, _ZSLAB)], isem).wait()

    return body


def _sc_gather_scatter(hw, src3, dst3):
    return _sc_gather_scatter_kernel()(hw, src3, dst3)


_BR = 2000


def _tc_prep(feats, norm, w0l, t1l, w0g, t1g):
    def body(f_ref, n_ref, wl_ref, tl_ref, wg_ref, tg_ref, hw_ref, x_ref):
        wl = jnp.dot(wl_ref[...], tl_ref[...], preferred_element_type=jnp.float32)
        wg = jnp.dot(wg_ref[...], tg_ref[...], preferred_element_type=jnp.float32)
        f = f_ref[...]
        hw_ref[...] = jnp.dot(f, wl, preferred_element_type=jnp.float32) * n_ref[...]
        x_ref[...] = jnp.dot(f, wg, preferred_element_type=jnp.float32)

    dd = pl.BlockSpec((D, D), lambda i: (0, 0))
    return pl.pallas_call(
        body,
        grid=(N // _BR,),
        in_specs=[
            pl.BlockSpec((_BR, D), lambda i: (i, 0)),
            pl.BlockSpec((_BR, 1), lambda i: (i, 0)),
            dd, dd, dd, dd,
        ],
        out_specs=[pl.BlockSpec((_BR, D), lambda i: (i, 0))] * 2,
        out_shape=[jax.ShapeDtypeStruct((N, D), jnp.float32)] * 2,
        compiler_params=pltpu.CompilerParams(dimension_semantics=("parallel",)),
    )(feats, norm, w0l, t1l, w0g, t1g)


_BP = 400


def _tc_ppmi1(ppmi, x1, w1g, t2g, bias):
    def body(p_ref, x_ref, wg2_ref, tg2_ref, b_ref, x2_ref):
        wg2 = jnp.dot(wg2_ref[...], tg2_ref[...],
                      preferred_element_type=jnp.float32)
        y1 = (jnp.dot(p_ref[...], x_ref[...], preferred_element_type=jnp.float32)
              + b_ref[...])
        x2_ref[...] = jnp.dot(y1, wg2, preferred_element_type=jnp.float32)

    dd = pl.BlockSpec((D, D), lambda i: (0, 0))
    return pl.pallas_call(
        body,
        grid=(N // _BP,),
        in_specs=[
            pl.BlockSpec((_BP, N), lambda i: (i, 0)),
            pl.BlockSpec((N, D), lambda i: (0, 0)),
            dd, dd,
            pl.BlockSpec((1, D), lambda i: (0, 0)),
        ],
        out_specs=pl.BlockSpec((_BP, D), lambda i: (i, 0)),
        out_shape=jax.ShapeDtypeStruct((N, D), jnp.float32),
        compiler_params=pltpu.CompilerParams(
            dimension_semantics=("arbitrary",)),
    )(ppmi, x1, w1g, t2g, bias)


def _tc_ppmi2(ppmi, x, bias):
    def body(p_ref, x_ref, b_ref, o_ref):
        o_ref[...] = (jnp.dot(p_ref[...], x_ref[...],
                              preferred_element_type=jnp.float32)
                      + b_ref[...])

    return pl.pallas_call(
        body,
        grid=(N // _BP,),
        in_specs=[
            pl.BlockSpec((_BP, N), lambda i: (i, 0)),
            pl.BlockSpec((N, D), lambda i: (0, 0)),
            pl.BlockSpec((1, D), lambda i: (0, 0)),
        ],
        out_specs=pl.BlockSpec((_BP, D), lambda i: (i, 0)),
        out_shape=jax.ShapeDtypeStruct((N, D), jnp.float32),
        compiler_params=pltpu.CompilerParams(
            dimension_semantics=("arbitrary",)),
    )(ppmi, x, bias)


def _tc_mid(agg, norm, b0l, w1l, t2l, xdep):
    def body(a_ref, n_ref, b_ref, wl_ref, tl_ref, xd_ref, hw_ref):
        wl = jnp.dot(wl_ref[...], tl_ref[...], preferred_element_type=jnp.float32)
        nrm = n_ref[...]
        h1 = (a_ref[0] + a_ref[1]) * nrm + b_ref[...]
        hw_ref[...] = jnp.dot(h1, wl, preferred_element_type=jnp.float32) * nrm

    dd = pl.BlockSpec((D, D), lambda i: (0, 0))
    return pl.pallas_call(
        body,
        grid=(N // _BR,),
        in_specs=[
            pl.BlockSpec((_NUM_CORES, _BR, D), lambda i: (0, i, 0)),
            pl.BlockSpec((_BR, 1), lambda i: (i, 0)),
            pl.BlockSpec((1, D), lambda i: (0, 0)),
            dd, dd,
            pl.BlockSpec((8, D), lambda i: (0, 0)),
        ],
        out_specs=pl.BlockSpec((_BR, D), lambda i: (i, 0)),
        out_shape=jax.ShapeDtypeStruct((N, D), jnp.float32),
        compiler_params=pltpu.CompilerParams(dimension_semantics=("parallel",)),
    )(agg, norm, b0l, w1l, t2l, xdep)


def _tc_fuse(agg, norm, b1l, y2, wal, wag, wc, bc):
    def body(a_ref, n_ref, b_ref, y_ref, wal_ref, wag_ref, wc_ref, bc_ref,
             o_ref):
        hl = (a_ref[0] + a_ref[1]) * n_ref[...] + b_ref[...]
        hg = y_ref[...]
        logits = (jnp.dot(hl, wal_ref[...], preferred_element_type=jnp.float32)
                  + jnp.dot(hg, wag_ref[...], preferred_element_type=jnp.float32))
        m = jnp.max(logits, axis=1, keepdims=True)
        e = jnp.exp(logits - m)
        a = e / jnp.sum(e, axis=1, keepdims=True)
        z = a[:, 0:1] * hl + a[:, 1:2] * hg
        o_ref[...] = (jnp.dot(z, wc_ref[...], preferred_element_type=jnp.float32)
                      + bc_ref[...])

    return pl.pallas_call(
        body,
        grid=(N // _BR,),
        in_specs=[
            pl.BlockSpec((_NUM_CORES, _BR, D), lambda i: (0, i, 0)),
            pl.BlockSpec((_BR, 1), lambda i: (i, 0)),
            pl.BlockSpec((1, D), lambda i: (0, 0)),
            pl.BlockSpec((_BR, D), lambda i: (i, 0)),
            pl.BlockSpec((D, 2), lambda i: (0, 0)),
            pl.BlockSpec((D, 2), lambda i: (0, 0)),
            pl.BlockSpec((D, NCLS), lambda i: (0, 0)),
            pl.BlockSpec((1, NCLS), lambda i: (0, 0)),
        ],
        out_specs=pl.BlockSpec((_BR, NCLS), lambda i: (i, 0)),
        out_shape=jax.ShapeDtypeStruct((N, NCLS), jnp.float32),
        compiler_params=pltpu.CompilerParams(dimension_semantics=("parallel",)),
    )(agg, norm, b1l, y2, wal, wag, wc, bc)


def kernel(feats, edge_index, norm, tao_1_L, tao_2_L, tao_1_G, tao_2_G, PPMI,
           w0L, b0L, w1L, b1L, w0G, b0G, w1G, b1G, W_a, W_c, b_c):
    pad = ((0, 0), (0, _SLABPAD - _SLABCH), (0, 0))
    src3 = jnp.pad(edge_index[0].reshape(_NW * _NSG, _SLABCH, _CHUNK), pad)
    dst3 = jnp.pad(edge_index[1].reshape(_NW * _NSG, _SLABCH, _CHUNK), pad)

    hw1p, x1 = _tc_prep(feats, norm, w0L, tao_1_L, w0G, tao_1_G)
    agg1 = _sc_gather_scatter(hw1p, src3, dst3)
    x2 = _tc_ppmi1(PPMI, x1, w1G, tao_2_G, b0G.reshape(1, D))
    hw2p = _tc_mid(agg1, norm, b0L.reshape(1, D), w1L, tao_2_L, x2)
    agg2 = _sc_gather_scatter(hw2p, src3, dst3)
    y2 = _tc_ppmi2(PPMI, x2, b1G.reshape(1, D))
    return _tc_fuse(agg2, norm, b1L.reshape(1, D), y2,
                    W_a[:D], W_a[D:], W_c, b_c.reshape(1, NCLS))
